# baseline (speedup 1.0000x reference)
import functools

import jax
import jax.numpy as jnp
from jax import lax
from jax.experimental import pallas as pl
from jax.experimental.pallas import tpu as pltpu
from jax.experimental.pallas import tpu_sc as plsc

VOCAB = 1000000
MAXLEN = 200
EMBED = 64
BATCH = 1024
SPLIT = 500096

_info = plsc.get_sparse_core_info()
_NC, _NS, _L = _info.num_cores, _info.num_subcores, _info.num_lanes
_NW = _NC * _NS


def _build(B, L, E):
    assert B % _NW == 0 and E % _L == 0
    rows_per_w = B // _NW
    assert rows_per_w % 2 == 0
    mesh = plsc.VectorSubcoreMesh(core_axis_name="c", subcore_axis_name="s")
    n_idx = rows_per_w * L

    @functools.partial(
        pl.kernel,
        mesh=mesh,
        compiler_params=pltpu.CompilerParams(use_tc_tiling_on_sc=False),
        out_type=jax.ShapeDtypeStruct((B, L, 2 * E), jnp.float32),
        scratch_types=[
            pltpu.VMEM((n_idx,), jnp.int32),
            pltpu.VMEM((n_idx,), jnp.int32),
            pltpu.VMEM((L, E), jnp.float32),
            pltpu.VMEM((L, E), jnp.float32),
            pltpu.VMEM((L, E), jnp.float32),
            pltpu.SemaphoreType.DMA,
            pltpu.SemaphoreType.DMA,
            pltpu.SemaphoreType.DMA,
            pltpu.SemaphoreType.DMA,
        ],
    )
    def k(x_hbm, tok_hbm, pos_hbm, out_hbm, idx_all, row_all, rows0, rows1,
          pos_v, gsem0, gsem1, osem0, osem1):
        wid = lax.axis_index("s") * _NC + lax.axis_index("c")
        base = wid * rows_per_w
        bufs = (rows0, rows1)
        gsems = (gsem0, gsem1)
        osems = (osem0, osem1)

        pltpu.sync_copy(x_hbm.at[pl.ds(base * L, n_idx)], idx_all)
        pltpu.sync_copy(pos_hbm, pos_v)

        @plsc.parallel_loop(0, n_idx // _L, unroll=4)
        def to_rows(c):
            sl = pl.ds(c * _L, _L)
            v = idx_all[sl]
            row_all[sl] = jnp.where(v < SPLIT, 2 * v, 2 * v - (2 * SPLIT - 1))

        def fire_gather(r, b):
            pltpu.async_copy(
                tok_hbm.at[row_all.at[pl.ds(r * L, 128)]],
                bufs[b].at[pl.ds(0, 128)], gsems[b])
            pltpu.async_copy(
                tok_hbm.at[row_all.at[pl.ds(r * L + 128, L - 128)]],
                bufs[b].at[pl.ds(128, L - 128)], gsems[b])

        def wait_gather(b):
            pltpu.make_async_copy(
                tok_hbm.at[row_all.at[pl.ds(0, 128)]],
                bufs[b].at[pl.ds(0, 128)], gsems[b]).wait()
            pltpu.make_async_copy(
                tok_hbm.at[row_all.at[pl.ds(0, L - 128)]],
                bufs[b].at[pl.ds(128, L - 128)], gsems[b]).wait()

        def wait_out(b):
            pltpu.make_async_copy(
                bufs[b], out_hbm.at[0, :, pl.ds(0, E)], osems[b]).wait()

        fire_gather(0, 0)
        fire_gather(1, 1)

        @pl.loop(0, rows_per_w, step=2)
        def per_pair(g):
            for b in range(2):
                r = g + b
                wait_gather(b)

                @plsc.parallel_loop(0, L, unroll=2)
                def add_pos(l):
                    for j in range(E // _L):
                        sl = pl.ds(j * _L, _L)
                        bufs[b][l, sl] = bufs[b][l, sl] + pos_v[l, sl]

                pltpu.async_copy(
                    bufs[b], out_hbm.at[base + r, :, pl.ds(0, E)], osems[b])

            @pl.when(g + 2 < rows_per_w)
            def _():
                for b in range(2):
                    wait_out(b)
                    fire_gather(g + 2 + b, b)

        wait_out(0)
        wait_out(1)

    return k


_emb = _build(BATCH, MAXLEN, EMBED)

_TP_VB = 128
_TP_NB = SPLIT // _TP_VB
_TP_LAST = VOCAB // _TP_VB


def _tp_body(a_ref, b_ref, o_ref):
    o_ref[...] = jnp.concatenate(
        [jnp.transpose(a_ref[...], (1, 0)), jnp.transpose(b_ref[...], (1, 0))],
        axis=1)


_pack_table = pl.pallas_call(
    _tp_body,
    grid=(_TP_NB,),
    in_specs=[
        pl.BlockSpec((EMBED, _TP_VB), lambda i: (0, i)),
        pl.BlockSpec((EMBED, _TP_VB),
                     lambda i: (0, jnp.minimum(i + _TP_NB, _TP_LAST))),
    ],
    out_specs=pl.BlockSpec((_TP_VB, 2 * EMBED), lambda i: (i, 0)),
    out_shape=jax.ShapeDtypeStruct((SPLIT, 2 * EMBED), jnp.float32),
)


def kernel(x, token_table, pos_table):
    xf = x.reshape(-1).astype(jnp.int32)
    tt_t = token_table.T
    tt = _pack_table(tt_t, tt_t).reshape(2 * SPLIT, EMBED)
    out = _emb(xf, tt, pos_table)
    return out[:, :, :EMBED]


# speedup vs baseline: 6.6661x; 6.6661x over previous
import functools

import jax
import jax.numpy as jnp
from jax import lax
from jax.experimental import pallas as pl
from jax.experimental.pallas import tpu as pltpu
from jax.experimental.pallas import tpu_sc as plsc

VOCAB = 1000000
MAXLEN = 200
EMBED = 64
BATCH = 1024
SPLIT = 507904

_info = plsc.get_sparse_core_info()
_NC, _NS, _L = _info.num_cores, _info.num_subcores, _info.num_lanes
_NW = _NC * _NS


def _build(B, L, E):
    assert B % _NW == 0 and E % _L == 0
    rows_per_w = B // _NW
    assert rows_per_w % 2 == 0
    mesh = plsc.VectorSubcoreMesh(core_axis_name="c", subcore_axis_name="s")
    n_idx = rows_per_w * L

    @functools.partial(
        pl.kernel,
        mesh=mesh,
        compiler_params=pltpu.CompilerParams(use_tc_tiling_on_sc=False),
        out_type=jax.ShapeDtypeStruct((B, L, 2 * E), jnp.float32),
        scratch_types=[
            pltpu.VMEM((n_idx,), jnp.int32),
            pltpu.VMEM((n_idx,), jnp.int32),
            pltpu.VMEM((L, E), jnp.float32),
            pltpu.VMEM((L, E), jnp.float32),
            pltpu.VMEM((L, E), jnp.float32),
            pltpu.SemaphoreType.DMA,
            pltpu.SemaphoreType.DMA,
            pltpu.SemaphoreType.DMA,
            pltpu.SemaphoreType.DMA,
        ],
    )
    def k(x_hbm, tok_hbm, pos_hbm, out_hbm, idx_all, row_all, rows0, rows1,
          pos_v, gsem0, gsem1, osem0, osem1):
        wid = lax.axis_index("s") * _NC + lax.axis_index("c")
        base = wid * rows_per_w
        bufs = (rows0, rows1)
        gsems = (gsem0, gsem1)
        osems = (osem0, osem1)

        pltpu.sync_copy(x_hbm.at[pl.ds(base * L, n_idx)], idx_all)
        pltpu.sync_copy(pos_hbm, pos_v)

        @plsc.parallel_loop(0, n_idx // _L, unroll=4)
        def to_rows(c):
            sl = pl.ds(c * _L, _L)
            v = idx_all[sl]
            row_all[sl] = jnp.where(v < SPLIT, 2 * v, 2 * v - (2 * SPLIT - 1))

        def fire_gather(r, b):
            pltpu.async_copy(
                tok_hbm.at[row_all.at[pl.ds(r * L, 128)]],
                bufs[b].at[pl.ds(0, 128)], gsems[b])
            pltpu.async_copy(
                tok_hbm.at[row_all.at[pl.ds(r * L + 128, L - 128)]],
                bufs[b].at[pl.ds(128, L - 128)], gsems[b])

        def wait_gather(b):
            pltpu.make_async_copy(
                tok_hbm.at[row_all.at[pl.ds(0, 128)]],
                bufs[b].at[pl.ds(0, 128)], gsems[b]).wait()
            pltpu.make_async_copy(
                tok_hbm.at[row_all.at[pl.ds(0, L - 128)]],
                bufs[b].at[pl.ds(128, L - 128)], gsems[b]).wait()

        def wait_out(b):
            pltpu.make_async_copy(
                bufs[b], out_hbm.at[0, :, pl.ds(0, E)], osems[b]).wait()

        fire_gather(0, 0)
        fire_gather(1, 1)

        @pl.loop(0, rows_per_w, step=2)
        def per_pair(g):
            for b in range(2):
                r = g + b
                wait_gather(b)

                @plsc.parallel_loop(0, L, unroll=2)
                def add_pos(l):
                    for j in range(E // _L):
                        sl = pl.ds(j * _L, _L)
                        bufs[b][l, sl] = bufs[b][l, sl] + pos_v[l, sl]

                pltpu.async_copy(
                    bufs[b], out_hbm.at[base + r, :, pl.ds(0, E)], osems[b])

            @pl.when(g + 2 < rows_per_w)
            def _():
                for b in range(2):
                    wait_out(b)
                    fire_gather(g + 2 + b, b)

        wait_out(0)
        wait_out(1)

    return k


_emb = _build(BATCH, MAXLEN, EMBED)

_TP_VB = 8192
_TP_NB = SPLIT // _TP_VB
_TP_LAST = VOCAB // _TP_VB


def _tp_body(a_ref, b_ref, o_ref):
    o_ref[...] = jnp.concatenate(
        [jnp.transpose(a_ref[...], (1, 0)), jnp.transpose(b_ref[...], (1, 0))],
        axis=1)


_pack_table = pl.pallas_call(
    _tp_body,
    grid=(_TP_NB,),
    in_specs=[
        pl.BlockSpec((EMBED, _TP_VB), lambda i: (0, i)),
        pl.BlockSpec((EMBED, _TP_VB),
                     lambda i: (0, jnp.minimum(i + _TP_NB, _TP_LAST))),
    ],
    out_specs=pl.BlockSpec((_TP_VB, 2 * EMBED), lambda i: (i, 0)),
    out_shape=jax.ShapeDtypeStruct((SPLIT, 2 * EMBED), jnp.float32),
)


def kernel(x, token_table, pos_table):
    xf = x.reshape(-1).astype(jnp.int32)
    tt_t = token_table.T
    tt = _pack_table(tt_t, tt_t).reshape(2 * SPLIT, EMBED)
    out = _emb(xf, tt, pos_table)
    return out[:, :, :EMBED]


# speedup vs baseline: 8.1010x; 1.2153x over previous
import functools

import jax
import jax.numpy as jnp
from jax import lax
from jax.experimental import pallas as pl
from jax.experimental.pallas import tpu as pltpu
from jax.experimental.pallas import tpu_sc as plsc

VOCAB = 1000000
MAXLEN = 200
EMBED = 64
BATCH = 1024
SPLIT = 507904

_info = plsc.get_sparse_core_info()
_NC, _NS, _L = _info.num_cores, _info.num_subcores, _info.num_lanes
_NW = _NC * _NS


def _build(B, L, E):
    assert B % _NW == 0 and E % _L == 0
    rows_per_w = B // _NW
    assert rows_per_w % 2 == 0
    mesh = plsc.VectorSubcoreMesh(core_axis_name="c", subcore_axis_name="s")
    n_idx = rows_per_w * L

    @functools.partial(
        pl.kernel,
        mesh=mesh,
        compiler_params=pltpu.CompilerParams(use_tc_tiling_on_sc=False),
        out_type=jax.ShapeDtypeStruct((B, L, 2 * E), jnp.float32),
        scratch_types=[
            pltpu.VMEM((n_idx,), jnp.int32),
            pltpu.VMEM((n_idx,), jnp.int32),
            pltpu.VMEM((L, E), jnp.float32),
            pltpu.VMEM((L, E), jnp.float32),
            pltpu.VMEM((L, E), jnp.float32),
            pltpu.SemaphoreType.DMA,
            pltpu.SemaphoreType.DMA,
            pltpu.SemaphoreType.DMA,
            pltpu.SemaphoreType.DMA,
        ],
    )
    def k(x_hbm, tok_hbm, pos_hbm, out_hbm, idx_all, row_all, rows0, rows1,
          pos_v, gsem0, gsem1, osem0, osem1):
        wid = lax.axis_index("s") * _NC + lax.axis_index("c")
        base = wid * rows_per_w
        bufs = (rows0, rows1)
        gsems = (gsem0, gsem1)
        osems = (osem0, osem1)

        pltpu.sync_copy(x_hbm.at[pl.ds(base * L, n_idx)], idx_all)
        pltpu.sync_copy(pos_hbm, pos_v)

        @plsc.parallel_loop(0, n_idx // _L, unroll=4)
        def to_rows(c):
            sl = pl.ds(c * _L, _L)
            v = idx_all[sl]
            row_all[sl] = jnp.where(v < SPLIT, 2 * v, 2 * v - (2 * SPLIT - 1))

        def fire_gather(r, b):
            pltpu.async_copy(
                tok_hbm.at[row_all.at[pl.ds(r * L, 128)]],
                bufs[b].at[pl.ds(0, 128)], gsems[b])
            pltpu.async_copy(
                tok_hbm.at[row_all.at[pl.ds(r * L + 128, L - 128)]],
                bufs[b].at[pl.ds(128, L - 128)], gsems[b])

        def wait_gather(b):
            pltpu.make_async_copy(
                tok_hbm.at[row_all.at[pl.ds(0, 128)]],
                bufs[b].at[pl.ds(0, 128)], gsems[b]).wait()
            pltpu.make_async_copy(
                tok_hbm.at[row_all.at[pl.ds(0, L - 128)]],
                bufs[b].at[pl.ds(128, L - 128)], gsems[b]).wait()

        def wait_out(b):
            pltpu.make_async_copy(
                bufs[b], out_hbm.at[0, :, pl.ds(0, E)], osems[b]).wait()

        fire_gather(0, 0)
        fire_gather(1, 1)

        @pl.loop(0, rows_per_w, step=2)
        def per_pair(g):
            for b in range(2):
                r = g + b
                wait_gather(b)

                @plsc.parallel_loop(0, L, unroll=2)
                def add_pos(l):
                    for j in range(E // _L):
                        sl = pl.ds(j * _L, _L)
                        bufs[b][l, sl] = bufs[b][l, sl] + pos_v[l, sl]

                pltpu.async_copy(
                    bufs[b], out_hbm.at[base + r, :, pl.ds(0, E)], osems[b])

            @pl.when(g + 2 < rows_per_w)
            def _():
                for b in range(2):
                    wait_out(b)
                    fire_gather(g + 2 + b, b)

        wait_out(0)
        wait_out(1)

    return k


_emb = _build(BATCH, MAXLEN, EMBED)

_TP_VB = 16384
_TP_NB = SPLIT // _TP_VB
_TP_LAST = VOCAB // _TP_VB


def _tp_body(a_ref, b_ref, o_ref):
    o_ref[...] = jnp.transpose(
        jnp.concatenate([a_ref[...], b_ref[...]], axis=0), (1, 0))


_pack_table = pl.pallas_call(
    _tp_body,
    grid=(_TP_NB,),
    in_specs=[
        pl.BlockSpec((EMBED, _TP_VB), lambda i: (0, i)),
        pl.BlockSpec((EMBED, _TP_VB),
                     lambda i: (0, jnp.minimum(i + _TP_NB, _TP_LAST))),
    ],
    out_specs=pl.BlockSpec((_TP_VB, 2 * EMBED), lambda i: (i, 0)),
    out_shape=jax.ShapeDtypeStruct((SPLIT, 2 * EMBED), jnp.float32),
)


def kernel(x, token_table, pos_table):
    xf = x.reshape(-1).astype(jnp.int32)
    tt_t = token_table.T
    tt = _pack_table(tt_t, tt_t).reshape(2 * SPLIT, EMBED)
    out = _emb(xf, tt, pos_table)
    return out[:, :, :EMBED]
